# Initial kernel scaffold; baseline (speedup 1.0000x reference)
import jax
import jax.numpy as jnp
from jax.experimental import pallas as pl


def kernel(x, pos, edge_index, edge_attr, batch, mu_r_norm, params):
    raise NotImplementedError("write your pallas kernel here")


# speedup vs baseline: 3.0505x; 3.0505x over previous
import functools

import jax
import jax.numpy as jnp
from jax import lax
from jax.experimental import pallas as pl
from jax.experimental.pallas import tpu as pltpu
from jax.experimental.pallas import tpu_sc as plsc

_N = 100000
_E = 1600000
_EP = 1638400
_NG = 64
_RB_N = 2000
_RB_E = 8192
_IDXR = _EP // 128
_NC, _NS = 2, 16
_NW = _NC * _NS
_NHALF = _N // 2
_SPROWS = 50048
_DUMMY = _NHALF
f32 = jnp.float32

_GROWS = _IDXR // _NW
_GSUP = 8
_GNS = _GROWS // _GSUP

_SROWS_PER_S = _IDXR // _NS
_SSUP = 4
_SNS = _SROWS_PER_S // _SSUP
_ZCH = _SSUP * 128
_ZB = _SPROWS // _NS
_WB = 3120
_WTAIL_GROUPS = (_NHALF - _WB * _NS) // 8


def _silu(v):
    return v * (1.0 / (1.0 + jnp.exp(-v)))


def _emb_body(x_ref, mu_ref, w1_ref, w2_ref, b_ref, o_ref):
    acc = jnp.dot(x_ref[...], w1_ref[...], preferred_element_type=f32)
    acc = acc + jnp.dot(mu_ref[...], w2_ref[...], preferred_element_type=f32)
    o_ref[...] = acc + b_ref[...]


def _emb(x, mu8, w1, w2, b):
    return pl.pallas_call(
        _emb_body,
        grid=(_N // _RB_N,),
        in_specs=[
            pl.BlockSpec((_RB_N, 128), lambda i: (i, 0)),
            pl.BlockSpec((_RB_N, 8), lambda i: (i, 0)),
            pl.BlockSpec((128, 32), lambda i: (0, 0)),
            pl.BlockSpec((8, 32), lambda i: (0, 0)),
            pl.BlockSpec((1, 32), lambda i: (0, 0)),
        ],
        out_specs=pl.BlockSpec((_RB_N, 32), lambda i: (i, 0)),
        out_shape=jax.ShapeDtypeStruct((_N, 32), f32),
    )(x, mu8, w1, w2, b)


def _proj_body(f_ref, wd_ref, ws_ref, pd_ref, ps_ref):
    f = f_ref[...]
    pd_ref[...] = jnp.dot(f, wd_ref[...], preferred_element_type=f32)
    ps_ref[...] = jnp.dot(f, ws_ref[...], preferred_element_type=f32)


def _proj(feats, wd, ws):
    return pl.pallas_call(
        _proj_body,
        grid=(_N // _RB_N,),
        in_specs=[
            pl.BlockSpec((_RB_N, 32), lambda i: (i, 0)),
            pl.BlockSpec((32, 32), lambda i: (0, 0)),
            pl.BlockSpec((32, 32), lambda i: (0, 0)),
        ],
        out_specs=[
            pl.BlockSpec((_RB_N, 32), lambda i: (i, 0)),
            pl.BlockSpec((_RB_N, 32), lambda i: (i, 0)),
        ],
        out_shape=[
            jax.ShapeDtypeStruct((_N, 32), f32),
            jax.ShapeDtypeStruct((_N, 32), f32),
        ],
    )(feats, wd, ws)


def _ef_body(pd_ref, ps_ref, ea_ref, f_ref):
    d = pd_ref[...] - ps_ref[...]
    rd = jnp.sum(d * d, axis=1, keepdims=True)
    f_ref[...] = jnp.concatenate(
        [ea_ref[...], rd, jnp.zeros((ea_ref.shape[0], 3), f32)], axis=1)


def _edgefeat(pd16, ps16, edge_attr_p):
    return pl.pallas_call(
        _ef_body,
        grid=(_EP // _RB_E,),
        in_specs=[
            pl.BlockSpec((_RB_E, 16), lambda i: (i, 0)),
            pl.BlockSpec((_RB_E, 16), lambda i: (i, 0)),
            pl.BlockSpec((_RB_E, 4), lambda i: (i, 0)),
        ],
        out_specs=pl.BlockSpec((_RB_E, 8), lambda i: (i, 0)),
        out_shape=jax.ShapeDtypeStruct((_EP, 8), f32),
    )(pd16, ps16, edge_attr_p)


def _edge_body(gd_ref, gs_ref, fe_ref, wf_ref, b1_ref, w2_ref, b2_ref, o_ref):
    pre = (gd_ref[...] + gs_ref[...]
           + jnp.dot(fe_ref[...], wf_ref[...], preferred_element_type=f32)
           + b1_ref[...])
    m1 = _silu(pre)
    o_ref[...] = _silu(jnp.dot(m1, w2_ref[...], preferred_element_type=f32)
                       + b2_ref[...])


def _edge_mlp(gd, gs, fe, wf, be1, we2, be2):
    return pl.pallas_call(
        _edge_body,
        grid=(_EP // _RB_E,),
        in_specs=[
            pl.BlockSpec((_RB_E, 32), lambda i: (i, 0)),
            pl.BlockSpec((_RB_E, 32), lambda i: (i, 0)),
            pl.BlockSpec((_RB_E, 8), lambda i: (i, 0)),
            pl.BlockSpec((8, 32), lambda i: (0, 0)),
            pl.BlockSpec((1, 32), lambda i: (0, 0)),
            pl.BlockSpec((32, 32), lambda i: (0, 0)),
            pl.BlockSpec((1, 32), lambda i: (0, 0)),
        ],
        out_specs=pl.BlockSpec((_RB_E, 32), lambda i: (i, 0)),
        out_shape=jax.ShapeDtypeStruct((_EP, 32), f32),
    )(gd, gs, fe, wf, be1, we2, be2)


def _node_body(f_ref, a_ref, w1a_ref, w1b_ref, b1_ref, w2_ref, b2_ref, o_ref):
    f = f_ref[...]
    t = (jnp.dot(f, w1a_ref[...], preferred_element_type=f32)
         + jnp.dot(a_ref[...], w1b_ref[...], preferred_element_type=f32)
         + b1_ref[...])
    t = _silu(t)
    o_ref[...] = f + jnp.dot(t, w2_ref[...], preferred_element_type=f32) + b2_ref[...]


def _node(feats, agg, w1a, w1b, b1, w2, b2):
    return pl.pallas_call(
        _node_body,
        grid=(_N // _RB_N,),
        in_specs=[
            pl.BlockSpec((_RB_N, 32), lambda i: (i, 0)),
            pl.BlockSpec((_RB_N, 32), lambda i: (i, 0)),
            pl.BlockSpec((32, 32), lambda i: (0, 0)),
            pl.BlockSpec((32, 32), lambda i: (0, 0)),
            pl.BlockSpec((1, 32), lambda i: (0, 0)),
            pl.BlockSpec((32, 32), lambda i: (0, 0)),
            pl.BlockSpec((1, 32), lambda i: (0, 0)),
        ],
        out_specs=pl.BlockSpec((_RB_N, 32), lambda i: (i, 0)),
        out_shape=jax.ShapeDtypeStruct((_N, 32), f32),
    )(feats, agg, w1a, w1b, b1, w2, b2)


def _pool_body(f_ref, b_ref, o_ref, acc_ref):
    i = pl.program_id(0)

    @pl.when(i == 0)
    def _():
        acc_ref[...] = jnp.zeros_like(acc_ref)

    f = f_ref[...]
    fext = jnp.concatenate(
        [f, jnp.ones((_RB_N, 1), f32), jnp.zeros((_RB_N, 7), f32)], axis=1)
    ids = b_ref[...]
    lbl = lax.broadcasted_iota(jnp.int32, (_RB_N, _NG), 1)
    oh = (ids == lbl).astype(f32)
    acc_ref[...] += lax.dot_general(
        oh, fext, (((0,), (0,)), ((), ())), preferred_element_type=f32)

    @pl.when(i == _N // _RB_N - 1)
    def _():
        o_ref[...] = acc_ref[:, 0:32] / jnp.maximum(acc_ref[:, 32:33], 1.0)


def _pool(feats, batch2d):
    return pl.pallas_call(
        _pool_body,
        grid=(_N // _RB_N,),
        in_specs=[
            pl.BlockSpec((_RB_N, 32), lambda i: (i, 0)),
            pl.BlockSpec((_RB_N, 1), lambda i: (i, 0)),
        ],
        out_specs=pl.BlockSpec((_NG, 32), lambda i: (0, 0)),
        out_shape=jax.ShapeDtypeStruct((_NG, 32), f32),
        scratch_shapes=[pltpu.VMEM((_NG, 40), f32)],
    )(feats, batch2d)


def _head_body(xm_ref, w1_ref, b1_ref, w2_ref, b2_ref, o_ref):
    h = jnp.maximum(
        jnp.dot(xm_ref[...], w1_ref[...], preferred_element_type=f32)
        + b1_ref[...], 0.0)
    o_ref[...] = jnp.dot(h, w2_ref[...], preferred_element_type=f32) + b2_ref[...]


def _head(xm, w1, b1, w2, b2):
    return pl.pallas_call(
        _head_body,
        out_shape=jax.ShapeDtypeStruct((_NG, 20), f32),
    )(xm, w1, b1, w2, b2)


def _idxmap_body(d_ref, o_ref):
    c = pl.program_id(0)
    d = d_ref[...]
    lo = jnp.where(d < _NHALF, d, _DUMMY)
    hi = jnp.where((d >= _NHALF) & (d < _N), d - _NHALF, _DUMMY)
    o_ref[0] = jnp.where(c == 0, lo, hi)


def _idxmap(dst_r):
    return pl.pallas_call(
        _idxmap_body,
        grid=(2,),
        in_specs=[pl.BlockSpec((_IDXR, 128), lambda c: (0, 0))],
        out_specs=pl.BlockSpec((1, _IDXR, 128), lambda c: (c, 0, 0)),
        out_shape=jax.ShapeDtypeStruct((2, _IDXR, 128), jnp.int32),
    )(dst_r)


def _make_sc_gather(w, tab_rows):
    mesh = plsc.VectorSubcoreMesh(core_axis_name="c", subcore_axis_name="s",
                                  num_cores=_NC, num_subcores=_NS)

    @functools.partial(
        pl.kernel,
        out_type=jax.ShapeDtypeStruct((_EP, w), f32),
        mesh=mesh,
        scratch_types=[
            pltpu.VMEM((_GSUP, 128), jnp.int32),
            pltpu.VMEM((_GSUP * 128, w), f32),
            pltpu.SemaphoreType.DMA,
        ],
        compiler_params=pltpu.CompilerParams(use_tc_tiling_on_sc=False),
    )
    def k(tab_hbm, idx_hbm, out_hbm, idx_v, rows_v, sem):
        c = lax.axis_index("c")
        s = lax.axis_index("s")
        wid = s * _NC + c
        base = wid * _GROWS

        def body(j, carry):
            r0 = base + j * _GSUP
            pltpu.sync_copy(idx_hbm.at[pl.ds(r0, _GSUP)], idx_v)
            descs = [
                pltpu.async_copy(tab_hbm.at[idx_v.at[q]],
                                 rows_v.at[pl.ds(q * 128, 128)], sem)
                for q in range(_GSUP)
            ]
            for dd in descs:
                dd.wait()
            pltpu.sync_copy(rows_v, out_hbm.at[pl.ds(r0 * 128, _GSUP * 128)])
            return carry

        lax.fori_loop(0, _GNS, body, 0)

    return k


_get_sc_gather = functools.lru_cache(maxsize=None)(_make_sc_gather)


def _sc_gather16(tab, idx_r):
    return _get_sc_gather(16, _N)(tab, idx_r)


def _sc_gather32(tab, idx_r):
    return _get_sc_gather(32, _N)(tab, idx_r)


def _make_sc_scatter():
    mesh = plsc.VectorSubcoreMesh(core_axis_name="c", subcore_axis_name="s",
                                  num_cores=_NC, num_subcores=_NS)

    @functools.partial(
        pl.kernel,
        out_type=jax.ShapeDtypeStruct((_N, 32), f32),
        mesh=mesh,
        scratch_types=[
            pltpu.VMEM_SHARED((_SPROWS, 32), f32),
            pltpu.VMEM((_SSUP, 128), jnp.int32),
            pltpu.VMEM((_SSUP * 128, 32), f32),
            pltpu.SemaphoreType.DMA,
        ],
        compiler_params=pltpu.CompilerParams(use_tc_tiling_on_sc=False),
    )
    def k(m2_hbm, idx_hbm, zero_hbm, out_hbm, sp, idx_v, rows_v, sem):
        c = lax.axis_index("c")
        s = lax.axis_index("s")

        pltpu.sync_copy(zero_hbm, rows_v)
        zbase = s * _ZB
        nfull = _ZB // _ZCH
        for t in range(nfull):
            pltpu.sync_copy(rows_v, sp.at[pl.ds(zbase + t * _ZCH, _ZCH)])
        rem = _ZB - nfull * _ZCH
        pltpu.sync_copy(rows_v.at[pl.ds(0, rem)],
                        sp.at[pl.ds(zbase + nfull * _ZCH, rem)])
        plsc.subcore_barrier()

        def body(j, carry):
            r0 = s * _SROWS_PER_S + j * _SSUP
            pltpu.sync_copy(idx_hbm.at[c, pl.ds(r0, _SSUP)], idx_v)
            pltpu.sync_copy(m2_hbm.at[pl.ds(r0 * 128, _SSUP * 128)], rows_v)
            descs = [
                pltpu.async_copy(rows_v.at[pl.ds(q * 128, 128)],
                                 sp.at[idx_v.at[q]], sem, add=True)
                for q in range(_SSUP)
            ]
            for dd in descs:
                dd.wait()
            return carry

        lax.fori_loop(0, _SNS, body, 0)
        plsc.subcore_barrier()

        wbase = s * _WB
        nw = _WB // _ZCH
        for t in range(nw + 1):
            n = _ZCH if t < nw else _WB - nw * _ZCH
            off = t * _ZCH
            pltpu.sync_copy(sp.at[pl.ds(wbase + off, n)],
                            rows_v.at[pl.ds(0, n)])
            pltpu.sync_copy(rows_v.at[pl.ds(0, n)],
                            out_hbm.at[pl.ds(c * _NHALF + wbase + off, n)])

        @pl.when(s < _WTAIL_GROUPS)
        def _():
            off = _NS * _WB + s * 8
            pltpu.sync_copy(sp.at[pl.ds(off, 8)], rows_v.at[pl.ds(0, 8)])
            pltpu.sync_copy(rows_v.at[pl.ds(0, 8)],
                            out_hbm.at[pl.ds(c * _NHALF + off, 8)])

    return k


_get_sc_scatter = functools.lru_cache(maxsize=None)(_make_sc_scatter)


def _sc_scatter(m2, idxm, zeros_st):
    return _get_sc_scatter()(m2, idxm, zeros_st)


def kernel(x, pos, edge_index, edge_attr, batch, mu_r_norm, params):
    src = edge_index[0]
    dst = edge_index[1]
    npad = _EP - _E
    src_g = jnp.pad(src, (0, npad)).reshape(_IDXR, 128)
    dst_g = jnp.pad(dst, (0, npad)).reshape(_IDXR, 128)
    dst_m = jnp.pad(dst, (0, npad), constant_values=_N).reshape(_IDXR, 128)
    edge_attr_p = jnp.pad(edge_attr, ((0, npad), (0, 0)))
    pos_pad = jnp.pad(pos, ((0, 0), (0, 13)))
    mu8 = jnp.pad(mu_r_norm, ((0, 0), (0, 3)))
    p = params
    w1 = p["W_emb"][0:128]
    w2 = jnp.pad(p["W_emb"][128:133], ((0, 3), (0, 0)))

    feats = _emb(x, mu8, w1, w2, p["b_emb"].reshape(1, 32))

    pd16 = _sc_gather16(pos_pad, dst_g)
    ps16 = _sc_gather16(pos_pad, src_g)
    fe = _edgefeat(pd16, ps16, edge_attr_p)
    idxm = _idxmap(dst_m)
    zeros_st = jnp.zeros((_SSUP * 128, 32), f32)

    for l, lp in enumerate(p["layers"]):
        we1 = lp["We1"]
        wf = jnp.concatenate(
            [we1[64:68], (4.0 ** l) * we1[68:69], jnp.zeros((3, 32), f32)], axis=0)
        pd_t, ps_t = _proj(feats, we1[0:32], we1[32:64])
        gd = _sc_gather32(pd_t, dst_g)
        gs = _sc_gather32(ps_t, src_g)
        m2 = _edge_mlp(gd, gs, fe, wf, lp["be1"].reshape(1, 32),
                       lp["We2"], lp["be2"].reshape(1, 32))
        agg = _sc_scatter(m2, idxm, zeros_st)
        feats = _node(feats, agg, lp["Wn1"][0:32], lp["Wn1"][32:64],
                      lp["bn1"].reshape(1, 32), lp["Wn2"], lp["bn2"].reshape(1, 32))

    xm = _pool(feats, batch.reshape(_N, 1))
    logits = _head(xm, p["Wc1"], p["bc1"].reshape(1, 64),
                   p["Wc2"], p["bc2"].reshape(1, 20))
    return (logits, xm)


# speedup vs baseline: 3.5091x; 1.1503x over previous
import functools

import jax
import jax.numpy as jnp
from jax import lax
from jax.experimental import pallas as pl
from jax.experimental.pallas import tpu as pltpu
from jax.experimental.pallas import tpu_sc as plsc

_N = 100000
_E = 1600000
_EP = 1638400
_NG = 64
_RB_N = 2000
_RB_E = 8192
_IDXR = _EP // 128
_NC, _NS = 2, 16
_NW = _NC * _NS
_NHALF = _N // 2
_SPROWS = 50048
_DUMMY = _NHALF
f32 = jnp.float32

_GROWS = _IDXR // _NW
_GSUP = 8
_GNS = _GROWS // _GSUP

_SROWS_PER_S = _IDXR // _NS
_SSUP = 4
_SNS = _SROWS_PER_S // _SSUP
_ZCH = _SSUP * 128
_ZB = _SPROWS // _NS
_WB = 3120
_WTAIL_GROUPS = (_NHALF - _WB * _NS) // 8


def _silu(v):
    return v * (1.0 / (1.0 + jnp.exp(-v)))


def _emb_body(x_ref, mu_ref, w1_ref, w2_ref, b_ref, o_ref):
    acc = jnp.dot(x_ref[...], w1_ref[...], preferred_element_type=f32)
    acc = acc + jnp.dot(mu_ref[...], w2_ref[...], preferred_element_type=f32)
    o_ref[...] = acc + b_ref[...]


def _emb(x, mu8, w1, w2, b):
    return pl.pallas_call(
        _emb_body,
        grid=(_N // _RB_N,),
        in_specs=[
            pl.BlockSpec((_RB_N, 128), lambda i: (i, 0)),
            pl.BlockSpec((_RB_N, 8), lambda i: (i, 0)),
            pl.BlockSpec((128, 32), lambda i: (0, 0)),
            pl.BlockSpec((8, 32), lambda i: (0, 0)),
            pl.BlockSpec((1, 32), lambda i: (0, 0)),
        ],
        out_specs=pl.BlockSpec((_RB_N, 32), lambda i: (i, 0)),
        out_shape=jax.ShapeDtypeStruct((_N, 32), f32),
    )(x, mu8, w1, w2, b)


def _proj_body(f_ref, wd_ref, ws_ref, pd_ref, ps_ref):
    f = f_ref[...]
    pd_ref[...] = jnp.dot(f, wd_ref[...], preferred_element_type=f32)
    ps_ref[...] = jnp.dot(f, ws_ref[...], preferred_element_type=f32)


def _proj(feats, wd, ws):
    return pl.pallas_call(
        _proj_body,
        grid=(_N // _RB_N,),
        in_specs=[
            pl.BlockSpec((_RB_N, 32), lambda i: (i, 0)),
            pl.BlockSpec((32, 32), lambda i: (0, 0)),
            pl.BlockSpec((32, 32), lambda i: (0, 0)),
        ],
        out_specs=[
            pl.BlockSpec((_RB_N, 32), lambda i: (i, 0)),
            pl.BlockSpec((_RB_N, 32), lambda i: (i, 0)),
        ],
        out_shape=[
            jax.ShapeDtypeStruct((_N, 32), f32),
            jax.ShapeDtypeStruct((_N, 32), f32),
        ],
    )(feats, wd, ws)


def _rd_body(d_ref, o_ref):
    d = d_ref[...]
    o_ref[...] = jnp.sum(d * d, axis=1, keepdims=True)


def _rdk(d16):
    return pl.pallas_call(
        _rd_body,
        grid=(_EP // _RB_E,),
        in_specs=[pl.BlockSpec((_RB_E, 16), lambda i: (i, 0))],
        out_specs=pl.BlockSpec((_RB_E, 1), lambda i: (i, 0)),
        out_shape=jax.ShapeDtypeStruct((_EP, 1), f32),
    )(d16)


def _edge_body(g_ref, fe_ref, wf_ref, b1_ref, w2_ref, b2_ref, o_ref):
    pre = (g_ref[...]
           + jnp.dot(fe_ref[...], wf_ref[...], preferred_element_type=f32)
           + b1_ref[...])
    m1 = _silu(pre)
    o_ref[...] = _silu(jnp.dot(m1, w2_ref[...], preferred_element_type=f32)
                       + b2_ref[...])


def _edge_mlp(g, fe, wf, be1, we2, be2):
    return pl.pallas_call(
        _edge_body,
        grid=(_EP // _RB_E,),
        in_specs=[
            pl.BlockSpec((_RB_E, 32), lambda i: (i, 0)),
            pl.BlockSpec((_RB_E, 8), lambda i: (i, 0)),
            pl.BlockSpec((8, 32), lambda i: (0, 0)),
            pl.BlockSpec((1, 32), lambda i: (0, 0)),
            pl.BlockSpec((32, 32), lambda i: (0, 0)),
            pl.BlockSpec((1, 32), lambda i: (0, 0)),
        ],
        out_specs=pl.BlockSpec((_RB_E, 32), lambda i: (i, 0)),
        out_shape=jax.ShapeDtypeStruct((_EP, 32), f32),
    )(g, fe, wf, be1, we2, be2)


def _node_body(f_ref, a_ref, w1a_ref, w1b_ref, b1_ref, w2_ref, b2_ref, o_ref):
    f = f_ref[...]
    t = (jnp.dot(f, w1a_ref[...], preferred_element_type=f32)
         + jnp.dot(a_ref[...], w1b_ref[...], preferred_element_type=f32)
         + b1_ref[...])
    t = _silu(t)
    o_ref[...] = f + jnp.dot(t, w2_ref[...], preferred_element_type=f32) + b2_ref[...]


def _node(feats, agg, w1a, w1b, b1, w2, b2):
    return pl.pallas_call(
        _node_body,
        grid=(_N // _RB_N,),
        in_specs=[
            pl.BlockSpec((_RB_N, 32), lambda i: (i, 0)),
            pl.BlockSpec((_RB_N, 32), lambda i: (i, 0)),
            pl.BlockSpec((32, 32), lambda i: (0, 0)),
            pl.BlockSpec((32, 32), lambda i: (0, 0)),
            pl.BlockSpec((1, 32), lambda i: (0, 0)),
            pl.BlockSpec((32, 32), lambda i: (0, 0)),
            pl.BlockSpec((1, 32), lambda i: (0, 0)),
        ],
        out_specs=pl.BlockSpec((_RB_N, 32), lambda i: (i, 0)),
        out_shape=jax.ShapeDtypeStruct((_N, 32), f32),
    )(feats, agg, w1a, w1b, b1, w2, b2)


def _pool_body(f_ref, b_ref, o_ref, acc_ref):
    i = pl.program_id(0)

    @pl.when(i == 0)
    def _():
        acc_ref[...] = jnp.zeros_like(acc_ref)

    f = f_ref[...]
    fext = jnp.concatenate(
        [f, jnp.ones((_RB_N, 1), f32), jnp.zeros((_RB_N, 7), f32)], axis=1)
    ids = b_ref[...]
    lbl = lax.broadcasted_iota(jnp.int32, (_RB_N, _NG), 1)
    oh = (ids == lbl).astype(f32)
    acc_ref[...] += lax.dot_general(
        oh, fext, (((0,), (0,)), ((), ())), preferred_element_type=f32)

    @pl.when(i == _N // _RB_N - 1)
    def _():
        o_ref[...] = acc_ref[:, 0:32] / jnp.maximum(acc_ref[:, 32:33], 1.0)


def _pool(feats, batch2d):
    return pl.pallas_call(
        _pool_body,
        grid=(_N // _RB_N,),
        in_specs=[
            pl.BlockSpec((_RB_N, 32), lambda i: (i, 0)),
            pl.BlockSpec((_RB_N, 1), lambda i: (i, 0)),
        ],
        out_specs=pl.BlockSpec((_NG, 32), lambda i: (0, 0)),
        out_shape=jax.ShapeDtypeStruct((_NG, 32), f32),
        scratch_shapes=[pltpu.VMEM((_NG, 40), f32)],
    )(feats, batch2d)


def _head_body(xm_ref, w1_ref, b1_ref, w2_ref, b2_ref, o_ref):
    h = jnp.maximum(
        jnp.dot(xm_ref[...], w1_ref[...], preferred_element_type=f32)
        + b1_ref[...], 0.0)
    o_ref[...] = jnp.dot(h, w2_ref[...], preferred_element_type=f32) + b2_ref[...]


def _head(xm, w1, b1, w2, b2):
    return pl.pallas_call(
        _head_body,
        out_shape=jax.ShapeDtypeStruct((_NG, 20), f32),
    )(xm, w1, b1, w2, b2)


def _idxmap_body(d_ref, o_ref):
    c = pl.program_id(0)
    d = d_ref[...]
    lo = jnp.where(d < _NHALF, d, _DUMMY)
    hi = jnp.where((d >= _NHALF) & (d < _N), d - _NHALF, _DUMMY)
    o_ref[0] = jnp.where(c == 0, lo, hi)


def _idxmap(dst_r):
    return pl.pallas_call(
        _idxmap_body,
        grid=(2,),
        in_specs=[pl.BlockSpec((_IDXR, 128), lambda c: (0, 0))],
        out_specs=pl.BlockSpec((1, _IDXR, 128), lambda c: (c, 0, 0)),
        out_shape=jax.ShapeDtypeStruct((2, _IDXR, 128), jnp.int32),
    )(dst_r)


def _make_sc_gather_combine(w, sub):
    mesh = plsc.VectorSubcoreMesh(core_axis_name="c", subcore_axis_name="s",
                                  num_cores=_NC, num_subcores=_NS)
    nlan = w // 16

    @functools.partial(
        pl.kernel,
        out_type=jax.ShapeDtypeStruct((_EP, w), f32),
        mesh=mesh,
        scratch_types=[
            pltpu.VMEM((_GSUP, 128), jnp.int32),
            pltpu.VMEM((_GSUP, 128), jnp.int32),
            pltpu.VMEM((_GSUP * 128, w), f32),
            pltpu.VMEM((_GSUP * 128, w), f32),
            pltpu.SemaphoreType.DMA,
        ],
        compiler_params=pltpu.CompilerParams(use_tc_tiling_on_sc=False),
    )
    def k(tab_d_hbm, tab_s_hbm, idxd_hbm, idxs_hbm, out_hbm,
          idxd_v, idxs_v, rows_a, rows_b, sem):
        c = lax.axis_index("c")
        s = lax.axis_index("s")
        wid = s * _NC + c
        base = wid * _GROWS

        def body(j, carry):
            r0 = base + j * _GSUP
            pltpu.sync_copy(idxd_hbm.at[pl.ds(r0, _GSUP)], idxd_v)
            pltpu.sync_copy(idxs_hbm.at[pl.ds(r0, _GSUP)], idxs_v)
            descs = [
                pltpu.async_copy(tab_d_hbm.at[idxd_v.at[q]],
                                 rows_a.at[pl.ds(q * 128, 128)], sem)
                for q in range(_GSUP)
            ] + [
                pltpu.async_copy(tab_s_hbm.at[idxs_v.at[q]],
                                 rows_b.at[pl.ds(q * 128, 128)], sem)
                for q in range(_GSUP)
            ]
            for dd in descs:
                dd.wait()

            def addbody(r, carry2):
                b0 = r * 8
                for t in range(8):
                    for h in range(nlan):
                        a = rows_a[b0 + t, pl.ds(h * 16, 16)]
                        b = rows_b[b0 + t, pl.ds(h * 16, 16)]
                        rows_a[b0 + t, pl.ds(h * 16, 16)] = (a - b) if sub else (a + b)
                return carry2

            lax.fori_loop(0, (_GSUP * 128) // 8, addbody, 0)
            pltpu.sync_copy(rows_a, out_hbm.at[pl.ds(r0 * 128, _GSUP * 128)])
            return carry

        lax.fori_loop(0, _GNS, body, 0)

    return k


_get_sc_gc = functools.lru_cache(maxsize=None)(_make_sc_gather_combine)


def _sc_posdiff(pos_pad, dst_r, src_r):
    return _get_sc_gc(16, True)(pos_pad, pos_pad, dst_r, src_r)


def _sc_gather_add(tab_d, tab_s, dst_r, src_r):
    return _get_sc_gc(32, False)(tab_d, tab_s, dst_r, src_r)


def _make_sc_scatter():
    mesh = plsc.VectorSubcoreMesh(core_axis_name="c", subcore_axis_name="s",
                                  num_cores=_NC, num_subcores=_NS)

    @functools.partial(
        pl.kernel,
        out_type=jax.ShapeDtypeStruct((_N, 32), f32),
        mesh=mesh,
        scratch_types=[
            pltpu.VMEM_SHARED((_SPROWS, 32), f32),
            pltpu.VMEM((_SSUP, 128), jnp.int32),
            pltpu.VMEM((_SSUP * 128, 32), f32),
            pltpu.SemaphoreType.DMA,
        ],
        compiler_params=pltpu.CompilerParams(use_tc_tiling_on_sc=False),
    )
    def k(m2_hbm, idx_hbm, zero_hbm, out_hbm, sp, idx_v, rows_v, sem):
        c = lax.axis_index("c")
        s = lax.axis_index("s")

        pltpu.sync_copy(zero_hbm, rows_v)
        zbase = s * _ZB
        nfull = _ZB // _ZCH
        for t in range(nfull):
            pltpu.sync_copy(rows_v, sp.at[pl.ds(zbase + t * _ZCH, _ZCH)])
        rem = _ZB - nfull * _ZCH
        pltpu.sync_copy(rows_v.at[pl.ds(0, rem)],
                        sp.at[pl.ds(zbase + nfull * _ZCH, rem)])
        plsc.subcore_barrier()

        def body(j, carry):
            r0 = s * _SROWS_PER_S + j * _SSUP
            pltpu.sync_copy(idx_hbm.at[c, pl.ds(r0, _SSUP)], idx_v)
            pltpu.sync_copy(m2_hbm.at[pl.ds(r0 * 128, _SSUP * 128)], rows_v)
            descs = [
                pltpu.async_copy(rows_v.at[pl.ds(q * 128, 128)],
                                 sp.at[idx_v.at[q]], sem, add=True)
                for q in range(_SSUP)
            ]
            for dd in descs:
                dd.wait()
            return carry

        lax.fori_loop(0, _SNS, body, 0)
        plsc.subcore_barrier()

        wbase = s * _WB
        nw = _WB // _ZCH
        for t in range(nw + 1):
            n = _ZCH if t < nw else _WB - nw * _ZCH
            off = t * _ZCH
            pltpu.sync_copy(sp.at[pl.ds(wbase + off, n)],
                            rows_v.at[pl.ds(0, n)])
            pltpu.sync_copy(rows_v.at[pl.ds(0, n)],
                            out_hbm.at[pl.ds(c * _NHALF + wbase + off, n)])

        @pl.when(s < _WTAIL_GROUPS)
        def _():
            off = _NS * _WB + s * 8
            pltpu.sync_copy(sp.at[pl.ds(off, 8)], rows_v.at[pl.ds(0, 8)])
            pltpu.sync_copy(rows_v.at[pl.ds(0, 8)],
                            out_hbm.at[pl.ds(c * _NHALF + off, 8)])

    return k


_get_sc_scatter = functools.lru_cache(maxsize=None)(_make_sc_scatter)


def _sc_scatter(m2, idxm, zeros_st):
    return _get_sc_scatter()(m2, idxm, zeros_st)


def kernel(x, pos, edge_index, edge_attr, batch, mu_r_norm, params):
    src = edge_index[0]
    dst = edge_index[1]
    npad = _EP - _E
    src_g = jnp.pad(src, (0, npad)).reshape(_IDXR, 128)
    dst_g = jnp.pad(dst, (0, npad)).reshape(_IDXR, 128)
    dst_m = jnp.pad(dst, (0, npad), constant_values=_N).reshape(_IDXR, 128)
    edge_attr_p = jnp.pad(edge_attr, ((0, npad), (0, 0)))
    pos_pad = jnp.pad(pos, ((0, 0), (0, 13)))
    mu8 = jnp.pad(mu_r_norm, ((0, 0), (0, 3)))
    p = params
    w1 = p["W_emb"][0:128]
    w2 = jnp.pad(p["W_emb"][128:133], ((0, 3), (0, 0)))

    feats = _emb(x, mu8, w1, w2, p["b_emb"].reshape(1, 32))

    d16 = _sc_posdiff(pos_pad, dst_g, src_g)
    rd1 = _rdk(d16)
    fe = jnp.concatenate([edge_attr_p, rd1, jnp.zeros((_EP, 3), f32)], axis=1)
    idxm = _idxmap(dst_m)
    zeros_st = jnp.zeros((_SSUP * 128, 32), f32)

    for l, lp in enumerate(p["layers"]):
        we1 = lp["We1"]
        wf = jnp.concatenate(
            [we1[64:68], (4.0 ** l) * we1[68:69], jnp.zeros((3, 32), f32)], axis=0)
        pd_t, ps_t = _proj(feats, we1[0:32], we1[32:64])
        g = _sc_gather_add(pd_t, ps_t, dst_g, src_g)
        m2 = _edge_mlp(g, fe, wf, lp["be1"].reshape(1, 32),
                       lp["We2"], lp["be2"].reshape(1, 32))
        agg = _sc_scatter(m2, idxm, zeros_st)
        feats = _node(feats, agg, lp["Wn1"][0:32], lp["Wn1"][32:64],
                      lp["bn1"].reshape(1, 32), lp["Wn2"], lp["bn2"].reshape(1, 32))

    xm = _pool(feats, batch.reshape(_N, 1))
    logits = _head(xm, p["Wc1"], p["bc1"].reshape(1, 64),
                   p["Wc2"], p["bc2"].reshape(1, 20))
    return (logits, xm)
